# Initial kernel scaffold; baseline (speedup 1.0000x reference)
import jax
import jax.numpy as jnp
from jax.experimental import pallas as pl


def kernel(x, edge_index, labels, W1, b1, W2, b2, Wfc, bfc):
    raise NotImplementedError("write your pallas kernel here")


# speedup vs baseline: 7.2630x; 7.2630x over previous
import functools

import jax
import jax.numpy as jnp
import numpy as np
from jax import lax
from jax.experimental import pallas as pl
from jax.experimental.pallas import tpu as pltpu
from jax.experimental.pallas import tpu_sc as plsc

N = 10000
E = 320000
D = 128
NCLS = 40

NC = 2
NS = 16
NW = NC * NS
CHUNK = 128
KCH = 80
GRP = 40
NGRP = KCH // GRP
EPAD = NW * KCH * CHUNK
RDUM = 240
R = N + RDUM
RPT = R // NS
RB = 1000


def _zero_buf(buf, rows, width):
    @pl.loop(0, rows)
    def _(r):
        @pl.loop(0, width, step=16)
        def _(w):
            buf.at[pl.ds(r, 1), pl.ds(w, 16)][...] = jnp.zeros((1, 16), jnp.float32)


def _fill_ones(buf, rows, width):
    @pl.loop(0, rows)
    def _(r):
        @pl.loop(0, width, step=16)
        def _(w):
            buf.at[pl.ds(r, 1), pl.ds(w, 16)][...] = jnp.ones((1, 16), jnp.float32)


def _sc_readout(acc, out_h, c, s, width):
    nfull = (N // RPT)
    @pl.when(s < nfull)
    def _():
        pltpu.sync_copy(acc.at[pl.ds(s * RPT, RPT)],
                        out_h.at[c, pl.ds(s * RPT, RPT)])
    @pl.when(s == nfull)
    def _():
        pltpu.sync_copy(acc.at[pl.ds(nfull * RPT, N - nfull * RPT)],
                        out_h.at[c, pl.ds(nfull * RPT, N - nfull * RPT)])


def _sc_degree(dstp):
    mesh = plsc.VectorSubcoreMesh(core_axis_name="c", subcore_axis_name="s")

    @functools.partial(
        pl.kernel,
        out_type=jax.ShapeDtypeStruct((NC, N, 16), jnp.float32),
        mesh=mesh,
        scratch_types=[
            pltpu.VMEM((KCH, CHUNK), jnp.int32),
            pltpu.VMEM((CHUNK, 16), jnp.float32),
            pltpu.VMEM_SHARED((R, 16), jnp.float32),
        ],
    )
    def k(dst_h, out_h, dst_v, ones_v, acc):
        c = lax.axis_index("c")
        s = lax.axis_index("s")
        g = c * NS + s
        pltpu.sync_copy(dst_h.at[g], dst_v)
        _zero_buf(ones_v, CHUNK, 16)
        @pl.loop(0, RPT, step=CHUNK)
        def _(r0):
            pltpu.sync_copy(ones_v, acc.at[pl.ds(s * RPT + r0, CHUNK)])
        _fill_ones(ones_v, CHUNK, 16)
        plsc.subcore_barrier()
        @pl.loop(0, KCH)
        def _(j):
            pltpu.sync_copy(ones_v, acc.at[dst_v.at[j]], add=True)
        plsc.subcore_barrier()
        _sc_readout(acc, out_h, c, s, 16)

    return k(dstp)


def _sc_aggregate(table, srcp, dstp):
    mesh = plsc.VectorSubcoreMesh(core_axis_name="c", subcore_axis_name="s")

    @functools.partial(
        pl.kernel,
        out_type=jax.ShapeDtypeStruct((NC, N, D), jnp.float32),
        mesh=mesh,
        scratch_types=[
            pltpu.VMEM((GRP, CHUNK), jnp.int32),
            pltpu.VMEM((GRP, CHUNK), jnp.int32),
            pltpu.VMEM((CHUNK, D), jnp.float32),
            pltpu.VMEM((CHUNK, D), jnp.float32),
            pltpu.VMEM_SHARED((R, D), jnp.float32),
            pltpu.SemaphoreType.DMA,
            pltpu.SemaphoreType.DMA,
        ],
    )
    def k(tab_h, src_h, dst_h, out_h, src_v, dst_v, bufa, bufb, acc, sga, sgb):
        c = lax.axis_index("c")
        s = lax.axis_index("s")
        g = c * NS + s
        _zero_buf(bufa, CHUNK, D)
        @pl.loop(0, RPT, step=CHUNK)
        def _(r0):
            pltpu.sync_copy(bufa, acc.at[pl.ds(s * RPT + r0, CHUNK)])
        plsc.subcore_barrier()

        @pl.loop(0, NGRP)
        def _(grp):
            pltpu.sync_copy(src_h.at[g, pl.ds(grp * GRP, GRP)], src_v)
            pltpu.sync_copy(dst_h.at[g, pl.ds(grp * GRP, GRP)], dst_v)
            pltpu.async_copy(tab_h.at[src_v.at[0]], bufa, sga)
            @pl.loop(0, GRP, step=2)
            def _(j):
                pltpu.make_async_copy(tab_h.at[src_v.at[j]], bufa, sga).wait()
                pltpu.async_copy(tab_h.at[src_v.at[j + 1]], bufb, sgb)
                pltpu.sync_copy(bufa, acc.at[dst_v.at[j]], add=True)
                pltpu.make_async_copy(tab_h.at[src_v.at[j + 1]], bufb, sgb).wait()
                @pl.when(j + 2 < GRP)
                def _():
                    pltpu.async_copy(tab_h.at[src_v.at[j + 2]], bufa, sga)
                pltpu.sync_copy(bufb, acc.at[dst_v.at[j + 1]], add=True)
        plsc.subcore_barrier()
        _sc_readout(acc, out_h, c, s, D)

    return k(table, srcp, dstp)


def _dis_block(dacc):
    deg = dacc[0, :, 0:1] + dacc[1, :, 0:1]
    return lax.rsqrt(deg + 1.0)


def _tc_matmul_kernel(x_ref, w_ref, b_ref, o_ref):
    o_ref[...] = jnp.dot(x_ref[...], w_ref[...],
                         preferred_element_type=jnp.float32) + b_ref[...]


def _tc_scale_kernel(h_ref, dacc_ref, o_ref):
    o_ref[...] = h_ref[...] * _dis_block(dacc_ref[...])


def _tc_layer_kernel(p_ref, hs_ref, dacc_ref, w_ref, b_ref, o_ref):
    dis = _dis_block(dacc_ref[...])
    agg = dis * (p_ref[0] + p_ref[1] + hs_ref[...])
    h = jnp.where(agg >= 0.0, agg, 0.01 * agg)
    o_ref[...] = (jnp.dot(h, w_ref[...],
                          preferred_element_type=jnp.float32) + b_ref[...]) * dis


def _tc_loss_kernel(p_ref, hs_ref, dacc_ref, w_ref, b_ref, lab_ref, o_ref):
    i = pl.program_id(0)
    dis = _dis_block(dacc_ref[...])
    agg = dis * (p_ref[0] + p_ref[1] + hs_ref[...])
    h = jnp.where(agg >= 0.0, agg, 0.01 * agg)
    logits = jnp.dot(h, w_ref[...], preferred_element_type=jnp.float32) + b_ref[...]
    m = jnp.max(logits, axis=1, keepdims=True)
    lse = m + jnp.log(jnp.sum(jnp.exp(logits - m), axis=1, keepdims=True))
    col = lax.broadcasted_iota(jnp.int32, logits.shape, 1)
    sel = jnp.sum(jnp.where(col == lab_ref[...], logits, 0.0), axis=1,
                  keepdims=True)
    part = jnp.sum(lse - sel)

    @pl.when(i == 0)
    def _():
        o_ref[0, 0] = 0.0

    o_ref[0, 0] += part / N


def kernel(x, edge_index, labels, W1, b1, W2, b2, Wfc, bfc):
    src = edge_index[0]
    dst = edge_index[1]
    npad = EPAD - E
    pad_dst = jnp.asarray(N + (np.arange(npad) % RDUM), jnp.int32)
    srcp = jnp.concatenate([src, jnp.zeros((npad,), jnp.int32)])
    dstp = jnp.concatenate([dst, pad_dst])
    srcp = srcp.reshape(NW, KCH, CHUNK)
    dstp = dstp.reshape(NW, KCH, CHUNK)

    grid = (N // RB,)
    row_spec = pl.BlockSpec((RB, D), lambda i: (i, 0))
    p_spec = pl.BlockSpec((NC, RB, D), lambda i: (0, i, 0))
    dacc_spec = pl.BlockSpec((NC, RB, 16), lambda i: (0, i, 0))
    w_spec = pl.BlockSpec((D, D), lambda i: (0, 0))
    b_spec = pl.BlockSpec((1, D), lambda i: (0, 0))
    hD = jax.ShapeDtypeStruct((N, D), jnp.float32)

    dacc = _sc_degree(dstp)

    h1_pre = pl.pallas_call(
        _tc_matmul_kernel, grid=grid,
        in_specs=[row_spec, w_spec, b_spec], out_specs=row_spec,
        out_shape=hD,
    )(x, W1, b1.reshape(1, D))

    hs1 = pl.pallas_call(
        _tc_scale_kernel, grid=grid,
        in_specs=[row_spec, dacc_spec], out_specs=row_spec,
        out_shape=hD,
    )(h1_pre, dacc)

    P1 = _sc_aggregate(hs1, srcp, dstp)

    hs2 = pl.pallas_call(
        _tc_layer_kernel, grid=grid,
        in_specs=[p_spec, row_spec, dacc_spec, w_spec, b_spec],
        out_specs=row_spec, out_shape=hD,
    )(P1, hs1, dacc, W2, b2.reshape(1, D))

    P2 = _sc_aggregate(hs2, srcp, dstp)

    wfc_p = jnp.zeros((D, D), jnp.float32).at[:, :NCLS].set(Wfc)
    bfc_p = jnp.full((1, D), -1e30, jnp.float32).at[0, :NCLS].set(bfc)

    loss = pl.pallas_call(
        _tc_loss_kernel, grid=grid,
        in_specs=[p_spec, row_spec, dacc_spec, w_spec, b_spec,
                  pl.BlockSpec((RB, 1), lambda i: (i, 0))],
        out_specs=pl.BlockSpec(memory_space=pltpu.SMEM),
        out_shape=jax.ShapeDtypeStruct((1, 1), jnp.float32),
    )(P2, hs2, dacc, wfc_p, bfc_p, labels.reshape(N, 1))

    return loss[0, 0]


# speedup vs baseline: 7.2730x; 1.0014x over previous
import functools

import jax
import jax.numpy as jnp
import numpy as np
from jax import lax
from jax.experimental import pallas as pl
from jax.experimental.pallas import tpu as pltpu
from jax.experimental.pallas import tpu_sc as plsc

N = 10000
E = 320000
D = 128
NCLS = 40

NC = 2
NS = 16
NW = NC * NS
CHUNK = 128
K0 = 40
K1 = 120
GRP = 40
EPAD = NS * (K0 + K1) * CHUNK
E0 = NS * K0 * CHUNK
RDUM = 240
R = N + RDUM
RPT = R // NS
RB = 1000


def _zero_buf(buf, rows, width):
    @pl.loop(0, rows)
    def _(r):
        @pl.loop(0, width, step=16)
        def _(w):
            buf.at[pl.ds(r, 1), pl.ds(w, 16)][...] = jnp.zeros((1, 16), jnp.float32)


def _fill_ones(buf, rows, width):
    @pl.loop(0, rows)
    def _(r):
        @pl.loop(0, width, step=16)
        def _(w):
            buf.at[pl.ds(r, 1), pl.ds(w, 16)][...] = jnp.ones((1, 16), jnp.float32)


def _sc_readout(acc, out_h, c, s, width):
    nfull = (N // RPT)
    @pl.when(s < nfull)
    def _():
        pltpu.sync_copy(acc.at[pl.ds(s * RPT, RPT)],
                        out_h.at[c, pl.ds(s * RPT, RPT)])
    @pl.when(s == nfull)
    def _():
        pltpu.sync_copy(acc.at[pl.ds(nfull * RPT, N - nfull * RPT)],
                        out_h.at[c, pl.ds(nfull * RPT, N - nfull * RPT)])


def _sc_degree(ones_hbm, dstp):
    kch = EPAD // (NW * CHUNK)
    mesh = plsc.VectorSubcoreMesh(core_axis_name="c", subcore_axis_name="s")

    @functools.partial(
        pl.kernel,
        out_type=jax.ShapeDtypeStruct((NC, N, D), jnp.float32),
        mesh=mesh,
        scratch_types=[
            pltpu.VMEM((kch, CHUNK), jnp.int32),
            pltpu.VMEM((CHUNK, D), jnp.float32),
            pltpu.VMEM_SHARED((R, D), jnp.float32),
        ],
    )
    def k(ones_h, dst_h, out_h, dst_v, ones_v, acc):
        c = lax.axis_index("c")
        s = lax.axis_index("s")
        g = c * NS + s
        pltpu.sync_copy(dst_h.at[g], dst_v)
        _zero_buf(ones_v, CHUNK, D)
        @pl.loop(0, RPT, step=CHUNK)
        def _(r0):
            pltpu.sync_copy(ones_v, acc.at[pl.ds(s * RPT + r0, CHUNK)])
        pltpu.sync_copy(ones_h, ones_v)
        plsc.subcore_barrier()
        @pl.loop(0, kch)
        def _(j):
            pltpu.sync_copy(ones_v, acc.at[dst_v.at[j]], add=True)
        plsc.subcore_barrier()
        _sc_readout(acc, out_h, c, s, D)

    return k(ones_hbm, dstp)


def _sc_aggregate(table, src0, dst0, src1, dst1, width=D):
    mesh = plsc.VectorSubcoreMesh(core_axis_name="c", subcore_axis_name="s")

    @functools.partial(
        pl.kernel,
        out_type=jax.ShapeDtypeStruct((NC, N, width), jnp.float32),
        mesh=mesh,
        scratch_types=[
            pltpu.VMEM((GRP, CHUNK), jnp.int32),
            pltpu.VMEM((GRP, CHUNK), jnp.int32),
            pltpu.VMEM((CHUNK, width), jnp.float32),
            pltpu.VMEM((CHUNK, width), jnp.float32),
            pltpu.VMEM_SHARED((R, width), jnp.float32),
            pltpu.SemaphoreType.DMA,
            pltpu.SemaphoreType.DMA,
        ],
    )
    def k(tab_h, src0_h, dst0_h, src1_h, dst1_h, out_h,
          src_v, dst_v, bufa, bufb, acc, sga, sgb):
        c = lax.axis_index("c")
        s = lax.axis_index("s")
        _zero_buf(bufa, CHUNK, width)
        @pl.loop(0, RPT, step=CHUNK)
        def _(r0):
            pltpu.sync_copy(bufa, acc.at[pl.ds(s * RPT + r0, CHUNK)])
        plsc.subcore_barrier()

        def agg_edges(src_h, dst_h, kch):
            @pl.loop(0, kch // GRP)
            def _(grp):
                pltpu.sync_copy(src_h.at[s, pl.ds(grp * GRP, GRP)], src_v)
                pltpu.sync_copy(dst_h.at[s, pl.ds(grp * GRP, GRP)], dst_v)
                pltpu.async_copy(tab_h.at[src_v.at[0]], bufa, sga)
                @pl.loop(0, GRP, step=2)
                def _(j):
                    pltpu.make_async_copy(tab_h.at[src_v.at[j]], bufa, sga).wait()
                    pltpu.async_copy(tab_h.at[src_v.at[j + 1]], bufb, sgb)
                    pltpu.sync_copy(bufa, acc.at[dst_v.at[j]], add=True)
                    pltpu.make_async_copy(
                        tab_h.at[src_v.at[j + 1]], bufb, sgb).wait()
                    @pl.when(j + 2 < GRP)
                    def _():
                        pltpu.async_copy(tab_h.at[src_v.at[j + 2]], bufa, sga)
                    pltpu.sync_copy(bufb, acc.at[dst_v.at[j + 1]], add=True)

        @pl.when(c == 0)
        def _():
            agg_edges(src0_h, dst0_h, K0)
        @pl.when(c == 1)
        def _():
            agg_edges(src1_h, dst1_h, K1)
        plsc.subcore_barrier()
        _sc_readout(acc, out_h, c, s, width)

    return k(table, src0, dst0, src1, dst1)


def _dis_block(dacc):
    deg = dacc[0, :, 0:1] + dacc[1, :, 0:1]
    return lax.rsqrt(deg + 1.0)


def _tc_matmul_kernel(x_ref, w_ref, b_ref, o_ref):
    o_ref[...] = jnp.dot(x_ref[...], w_ref[...],
                         preferred_element_type=jnp.float32) + b_ref[...]


def _tc_scale_kernel(h_ref, dacc_ref, o_ref):
    o_ref[...] = h_ref[...] * _dis_block(dacc_ref[...])


def _tc_layer_kernel(p_ref, hs_ref, dacc_ref, w_ref, b_ref, o_ref):
    dis = _dis_block(dacc_ref[...])
    agg = dis * (p_ref[0] + p_ref[1] + hs_ref[...])
    h = jnp.where(agg >= 0.0, agg, 0.01 * agg)
    o_ref[...] = (jnp.dot(h, w_ref[...],
                          preferred_element_type=jnp.float32) + b_ref[...]) * dis


def _tc_loss_kernel(p_ref, hs_ref, dacc_ref, w_ref, b_ref, lab_ref, o_ref):
    i = pl.program_id(0)
    dis = _dis_block(dacc_ref[...])
    agg = dis * (p_ref[0] + p_ref[1] + hs_ref[...])
    h = jnp.where(agg >= 0.0, agg, 0.01 * agg)
    logits = jnp.dot(h, w_ref[...], preferred_element_type=jnp.float32) + b_ref[...]
    m = jnp.max(logits, axis=1, keepdims=True)
    lse = m + jnp.log(jnp.sum(jnp.exp(logits - m), axis=1, keepdims=True))
    col = lax.broadcasted_iota(jnp.int32, logits.shape, 1)
    sel = jnp.sum(jnp.where(col == lab_ref[...], logits, 0.0), axis=1,
                  keepdims=True)
    part = jnp.sum(lse - sel)

    @pl.when(i == 0)
    def _():
        o_ref[0, 0] = 0.0

    o_ref[0, 0] += part / N


def kernel(x, edge_index, labels, W1, b1, W2, b2, Wfc, bfc):
    src = edge_index[0]
    dst = edge_index[1]
    npad = EPAD - E
    pad_dst = jnp.asarray(N + (np.arange(npad) % RDUM), jnp.int32)
    srcp = jnp.concatenate([src, jnp.zeros((npad,), jnp.int32)])
    dstp = jnp.concatenate([dst, pad_dst])
    src0 = srcp[:E0].reshape(NS, K0, CHUNK)
    dst0 = dstp[:E0].reshape(NS, K0, CHUNK)
    src1 = srcp[E0:].reshape(NS, K1, CHUNK)
    dst1 = dstp[E0:].reshape(NS, K1, CHUNK)

    grid = (N // RB,)
    row_spec = pl.BlockSpec((RB, D), lambda i: (i, 0))
    p_spec = pl.BlockSpec((NC, RB, D), lambda i: (0, i, 0))
    dacc_spec = pl.BlockSpec((NC, RB, D), lambda i: (0, i, 0))
    w_spec = pl.BlockSpec((D, D), lambda i: (0, 0))
    b_spec = pl.BlockSpec((1, D), lambda i: (0, 0))
    hD = jax.ShapeDtypeStruct((N, D), jnp.float32)

    dacc = _sc_degree(jnp.ones((CHUNK, D), jnp.float32),
                      dstp.reshape(NW, EPAD // (NW * CHUNK), CHUNK))

    h1_pre = pl.pallas_call(
        _tc_matmul_kernel, grid=grid,
        in_specs=[row_spec, w_spec, b_spec], out_specs=row_spec,
        out_shape=hD,
    )(x, W1, b1.reshape(1, D))

    hs1 = pl.pallas_call(
        _tc_scale_kernel, grid=grid,
        in_specs=[row_spec, dacc_spec], out_specs=row_spec,
        out_shape=hD,
    )(h1_pre, dacc)

    P1 = _sc_aggregate(hs1, src0, dst0, src1, dst1)

    hs2 = pl.pallas_call(
        _tc_layer_kernel, grid=grid,
        in_specs=[p_spec, row_spec, dacc_spec, w_spec, b_spec],
        out_specs=row_spec, out_shape=hD,
    )(P1, hs1, dacc, W2, b2.reshape(1, D))

    P2 = _sc_aggregate(hs2, src0, dst0, src1, dst1)

    wfc_p = jnp.zeros((D, D), jnp.float32).at[:, :NCLS].set(Wfc)
    bfc_p = jnp.full((1, D), -1e30, jnp.float32).at[0, :NCLS].set(bfc)

    loss = pl.pallas_call(
        _tc_loss_kernel, grid=grid,
        in_specs=[p_spec, row_spec, dacc_spec, w_spec, b_spec,
                  pl.BlockSpec((RB, 1), lambda i: (i, 0))],
        out_specs=pl.BlockSpec(memory_space=pltpu.SMEM),
        out_shape=jax.ShapeDtypeStruct((1, 1), jnp.float32),
    )(P2, hs2, dacc, wfc_p, bfc_p, labels.reshape(N, 1))

    return loss[0, 0]


# speedup vs baseline: 7.2770x; 1.0006x over previous
import functools

import jax
import jax.numpy as jnp
import numpy as np
from jax import lax
from jax.experimental import pallas as pl
from jax.experimental.pallas import tpu as pltpu
from jax.experimental.pallas import tpu_sc as plsc

N = 10000
E = 320000
D = 128
NCLS = 40

NC = 2
NS = 16
NW = NC * NS
CHUNK = 128
K0 = 40
K1 = 120
GRP = 40
SUB = 2
HS = CHUNK // SUB
EPAD = NS * (K0 + K1) * CHUNK
E0 = NS * K0 * CHUNK
RDUM = 240
R = N + RDUM
RPT = R // NS
RB = 1000


def _zero_buf(buf, rows, width):
    @pl.loop(0, rows)
    def _(r):
        @pl.loop(0, width, step=16)
        def _(w):
            buf.at[pl.ds(r, 1), pl.ds(w, 16)][...] = jnp.zeros((1, 16), jnp.float32)


def _fill_ones(buf, rows, width):
    @pl.loop(0, rows)
    def _(r):
        @pl.loop(0, width, step=16)
        def _(w):
            buf.at[pl.ds(r, 1), pl.ds(w, 16)][...] = jnp.ones((1, 16), jnp.float32)


def _sc_readout(acc, out_h, c, s, width):
    nfull = (N // RPT)
    @pl.when(s < nfull)
    def _():
        pltpu.sync_copy(acc.at[pl.ds(s * RPT, RPT)],
                        out_h.at[c, pl.ds(s * RPT, RPT)])
    @pl.when(s == nfull)
    def _():
        pltpu.sync_copy(acc.at[pl.ds(nfull * RPT, N - nfull * RPT)],
                        out_h.at[c, pl.ds(nfull * RPT, N - nfull * RPT)])


def _sc_degree(ones_hbm, dstp):
    kch = EPAD // (NW * CHUNK)
    mesh = plsc.VectorSubcoreMesh(core_axis_name="c", subcore_axis_name="s")

    @functools.partial(
        pl.kernel,
        out_type=jax.ShapeDtypeStruct((NC, N, D), jnp.float32),
        mesh=mesh,
        scratch_types=[
            pltpu.VMEM((kch, CHUNK), jnp.int32),
            pltpu.VMEM((CHUNK, D), jnp.float32),
            pltpu.VMEM_SHARED((R, D), jnp.float32),
        ],
    )
    def k(ones_h, dst_h, out_h, dst_v, ones_v, acc):
        c = lax.axis_index("c")
        s = lax.axis_index("s")
        g = c * NS + s
        pltpu.sync_copy(dst_h.at[g], dst_v)
        _zero_buf(ones_v, CHUNK, D)
        @pl.loop(0, RPT, step=CHUNK)
        def _(r0):
            pltpu.sync_copy(ones_v, acc.at[pl.ds(s * RPT + r0, CHUNK)])
        pltpu.sync_copy(ones_h, ones_v)
        plsc.subcore_barrier()
        @pl.loop(0, kch)
        def _(j):
            pltpu.sync_copy(ones_v, acc.at[dst_v.at[j]], add=True)
        plsc.subcore_barrier()
        _sc_readout(acc, out_h, c, s, D)

    return k(ones_hbm, dstp)


def _sc_aggregate(table, src0, dst0, src1, dst1, width=D):
    mesh = plsc.VectorSubcoreMesh(core_axis_name="c", subcore_axis_name="s")

    @functools.partial(
        pl.kernel,
        out_type=jax.ShapeDtypeStruct((NC, N, width), jnp.float32),
        mesh=mesh,
        scratch_types=[
            pltpu.VMEM((GRP, CHUNK), jnp.int32),
            pltpu.VMEM((GRP, CHUNK), jnp.int32),
            pltpu.VMEM((CHUNK, width), jnp.float32),
            pltpu.VMEM((CHUNK, width), jnp.float32),
            pltpu.VMEM_SHARED((R, width), jnp.float32),
            pltpu.SemaphoreType.DMA,
            pltpu.SemaphoreType.DMA,
        ],
    )
    def k(tab_h, src0_h, dst0_h, src1_h, dst1_h, out_h,
          src_v, dst_v, bufa, bufb, acc, sga, sgb):
        c = lax.axis_index("c")
        s = lax.axis_index("s")
        _zero_buf(bufa, CHUNK, width)
        @pl.loop(0, RPT, step=CHUNK)
        def _(r0):
            pltpu.sync_copy(bufa, acc.at[pl.ds(s * RPT + r0, CHUNK)])
        plsc.subcore_barrier()

        def gissue(j, buf, sem):
            for h in range(SUB):
                pltpu.async_copy(tab_h.at[src_v.at[j, pl.ds(h * HS, HS)]],
                                 buf.at[pl.ds(h * HS, HS)], sem)

        def gwait(j, buf, sem):
            for h in range(SUB):
                pltpu.make_async_copy(
                    tab_h.at[src_v.at[j, pl.ds(h * HS, HS)]],
                    buf.at[pl.ds(h * HS, HS)], sem).wait()

        def agg_edges(src_h, dst_h, kch):
            @pl.loop(0, kch // GRP)
            def _(grp):
                pltpu.sync_copy(src_h.at[s, pl.ds(grp * GRP, GRP)], src_v)
                pltpu.sync_copy(dst_h.at[s, pl.ds(grp * GRP, GRP)], dst_v)
                gissue(0, bufa, sga)
                @pl.loop(0, GRP, step=2)
                def _(j):
                    gwait(j, bufa, sga)
                    gissue(j + 1, bufb, sgb)
                    pltpu.sync_copy(bufa, acc.at[dst_v.at[j]], add=True)
                    gwait(j + 1, bufb, sgb)
                    @pl.when(j + 2 < GRP)
                    def _():
                        gissue(j + 2, bufa, sga)
                    pltpu.sync_copy(bufb, acc.at[dst_v.at[j + 1]], add=True)

        @pl.when(c == 0)
        def _():
            agg_edges(src0_h, dst0_h, K0)
        @pl.when(c == 1)
        def _():
            agg_edges(src1_h, dst1_h, K1)
        plsc.subcore_barrier()
        _sc_readout(acc, out_h, c, s, width)

    return k(table, src0, dst0, src1, dst1)


def _dis_block(dacc):
    deg = dacc[0, :, 0:1] + dacc[1, :, 0:1]
    return lax.rsqrt(deg + 1.0)


def _tc_matmul_kernel(x_ref, w_ref, b_ref, o_ref):
    o_ref[...] = jnp.dot(x_ref[...], w_ref[...],
                         preferred_element_type=jnp.float32) + b_ref[...]


def _tc_scale_kernel(h_ref, dacc_ref, o_ref):
    o_ref[...] = h_ref[...] * _dis_block(dacc_ref[...])


def _tc_layer_kernel(p_ref, hs_ref, dacc_ref, w_ref, b_ref, o_ref):
    dis = _dis_block(dacc_ref[...])
    agg = dis * (p_ref[0] + p_ref[1] + hs_ref[...])
    h = jnp.where(agg >= 0.0, agg, 0.01 * agg)
    o_ref[...] = (jnp.dot(h, w_ref[...],
                          preferred_element_type=jnp.float32) + b_ref[...]) * dis


def _tc_loss_kernel(p_ref, hs_ref, dacc_ref, w_ref, b_ref, lab_ref, o_ref):
    i = pl.program_id(0)
    dis = _dis_block(dacc_ref[...])
    agg = dis * (p_ref[0] + p_ref[1] + hs_ref[...])
    h = jnp.where(agg >= 0.0, agg, 0.01 * agg)
    logits = jnp.dot(h, w_ref[...], preferred_element_type=jnp.float32) + b_ref[...]
    m = jnp.max(logits, axis=1, keepdims=True)
    lse = m + jnp.log(jnp.sum(jnp.exp(logits - m), axis=1, keepdims=True))
    col = lax.broadcasted_iota(jnp.int32, logits.shape, 1)
    sel = jnp.sum(jnp.where(col == lab_ref[...], logits, 0.0), axis=1,
                  keepdims=True)
    part = jnp.sum(lse - sel)

    @pl.when(i == 0)
    def _():
        o_ref[0, 0] = 0.0

    o_ref[0, 0] += part / N


def kernel(x, edge_index, labels, W1, b1, W2, b2, Wfc, bfc):
    src = edge_index[0]
    dst = edge_index[1]
    npad = EPAD - E
    pad_dst = jnp.asarray(N + (np.arange(npad) % RDUM), jnp.int32)
    srcp = jnp.concatenate([src, jnp.zeros((npad,), jnp.int32)])
    dstp = jnp.concatenate([dst, pad_dst])
    src0 = srcp[:E0].reshape(NS, K0, CHUNK)
    dst0 = dstp[:E0].reshape(NS, K0, CHUNK)
    src1 = srcp[E0:].reshape(NS, K1, CHUNK)
    dst1 = dstp[E0:].reshape(NS, K1, CHUNK)

    grid = (N // RB,)
    row_spec = pl.BlockSpec((RB, D), lambda i: (i, 0))
    p_spec = pl.BlockSpec((NC, RB, D), lambda i: (0, i, 0))
    dacc_spec = pl.BlockSpec((NC, RB, D), lambda i: (0, i, 0))
    w_spec = pl.BlockSpec((D, D), lambda i: (0, 0))
    b_spec = pl.BlockSpec((1, D), lambda i: (0, 0))
    hD = jax.ShapeDtypeStruct((N, D), jnp.float32)

    dacc = _sc_degree(jnp.ones((CHUNK, D), jnp.float32),
                      dstp.reshape(NW, EPAD // (NW * CHUNK), CHUNK))

    h1_pre = pl.pallas_call(
        _tc_matmul_kernel, grid=grid,
        in_specs=[row_spec, w_spec, b_spec], out_specs=row_spec,
        out_shape=hD,
    )(x, W1, b1.reshape(1, D))

    hs1 = pl.pallas_call(
        _tc_scale_kernel, grid=grid,
        in_specs=[row_spec, dacc_spec], out_specs=row_spec,
        out_shape=hD,
    )(h1_pre, dacc)

    P1 = _sc_aggregate(hs1, src0, dst0, src1, dst1)

    hs2 = pl.pallas_call(
        _tc_layer_kernel, grid=grid,
        in_specs=[p_spec, row_spec, dacc_spec, w_spec, b_spec],
        out_specs=row_spec, out_shape=hD,
    )(P1, hs1, dacc, W2, b2.reshape(1, D))

    P2 = _sc_aggregate(hs2, src0, dst0, src1, dst1)

    wfc_p = jnp.zeros((D, D), jnp.float32).at[:, :NCLS].set(Wfc)
    bfc_p = jnp.full((1, D), -1e30, jnp.float32).at[0, :NCLS].set(bfc)

    loss = pl.pallas_call(
        _tc_loss_kernel, grid=grid,
        in_specs=[p_spec, row_spec, dacc_spec, w_spec, b_spec,
                  pl.BlockSpec((RB, 1), lambda i: (i, 0))],
        out_specs=pl.BlockSpec(memory_space=pltpu.SMEM),
        out_shape=jax.ShapeDtypeStruct((1, 1), jnp.float32),
    )(P2, hs2, dacc, wfc_p, bfc_p, labels.reshape(N, 1))

    return loss[0, 0]
